# baseline (speedup 1.0000x reference)
import functools

import jax
import jax.numpy as jnp
from jax import lax
from jax.experimental import pallas as pl
from jax.experimental.pallas import tpu as pltpu
from jax.experimental.pallas import tpu_sc as plsc

H, W = 1024, 2048
CROP_H, CROP_W = 720, 1280
SAMPLES = 4
LABEL_COUNT = 19
NC, NS = 2, 16
NWORK = NC * NS
ACT = 30
RPW = CROP_H // ACT
SR = 32
WB = 1408
NVEC = WB // 16
NBANK = 4
CROP_PAD = 1280
HR = SAMPLES * CROP_PAD // 128
ROW_CLAMP = H - SR

_TOPS = (219, 196, 73, 29)
_LEFTS = (192, 367, 42, 696)

_mesh = plsc.VectorSubcoreMesh(core_axis_name="c", subcore_axis_name="s")
_params = pltpu.CompilerParams(
    use_tc_tiling_on_sc=True, needs_layout_passes=False)


def _pick(vec, iota, k):
    return jnp.max(jnp.where(iota == k, vec, 0))


def _col_masks(shift, iota_np):
    import numpy as np
    masks = []
    for j in range(NVEC):
        cols = iota_np + 16 * j
        m = (cols >= shift) & (cols < shift + CROP_W)
        masks.append(m)
    return masks


@functools.partial(
    pl.kernel,
    out_type=jax.ShapeDtypeStruct((ACT * SAMPLES * CROP_PAD,), jnp.int32),
    mesh=_mesh,
    scratch_types=[
        pltpu.VMEM((SR, WB), jnp.int32),
        pltpu.VMEM((SR, WB), jnp.int32),
        pltpu.VMEM((SAMPLES * CROP_PAD,), jnp.int32),
        pltpu.SemaphoreType.DMA,
        pltpu.SemaphoreType.DMA,
    ],
    compiler_params=_params,
)
def _hist_kernel(label_hbm, out_hbm, buf0_v, buf1_v, hist_v, sem0, sem1):
    import numpy as np
    w = lax.axis_index("s") * NC + lax.axis_index("c")
    iota = lax.iota(jnp.int32, 16)
    iota_np = np.arange(16)
    zeros = jnp.zeros((16,), jnp.int32)
    ones = jnp.ones((16,), jnp.int32)
    bufs = (buf0_v, buf1_v)
    sems = (sem0, sem1)

    @pl.when(w < ACT)
    def _():
        for k in range(SAMPLES * CROP_PAD // 16):
            hist_v[pl.ds(k * 16, 16)] = zeros

        def src_roff(c):
            top, left = _TOPS[c], _LEFTS[c]
            lb = min(left & -128, W - WB)
            base8 = (top & -8) + RPW * w
            start = pl.multiple_of(jnp.minimum(base8, ROW_CLAMP), 8)
            roff = base8 - start + (top & 7)
            return label_hbm.at[pl.ds(start, SR), lb:lb + WB], roff

        for c in range(SAMPLES):
            srcc, roffc = src_roff(c)
            pltpu.sync_copy(srcc, bufs[c % 2])
            roffs = {c: roffc}

            top, left = _TOPS[c], _LEFTS[c]
            lb = min(left & -128, W - WB)
            shift = left - lb
            masks_np = _col_masks(shift, iota_np)
            js = [j for j in range(NVEC) if masks_np[j].any()]
            buf_v = bufs[c % 2]
            roff = roffs[c]

            def body(i, carry, _c=c, _js=js, _masks=masks_np,
                     _roff=roff, _buf=buf_v, _shift=shift):
                r = _roff + i
                for g in range(0, len(_js), 8):
                    grp = _js[g:g + 8]
                    idxs, ms = [], []
                    for j in grp:
                        lv = _buf[r, pl.ds(j * 16, 16)]
                        bank = (j % NBANK) * 16
                        idxs.append(lv * (16 * NBANK)
                                    + (_c * CROP_PAD + bank + iota))
                        if _masks[j].all():
                            ms.append(None)
                        else:
                            cols16 = iota + (16 * j)
                            ms.append(jnp.logical_and(
                                cols16 >= _shift, cols16 < _shift + CROP_W))
                    for k in range(len(grp)):
                        plsc.addupdate_scatter(
                            hist_v, [idxs[k]], ones, mask=ms[k])
                return carry

            lax.fori_loop(0, RPW, body, 0)

        pltpu.sync_copy(
            hist_v,
            out_hbm.at[pl.ds(
                pl.multiple_of(SAMPLES * CROP_PAD * w, 8),
                SAMPLES * CROP_PAD)])


@functools.partial(
    pl.kernel,
    out_type=(
        jax.ShapeDtypeStruct((3, CROP_H, CROP_W), jnp.float32),
        jax.ShapeDtypeStruct((CROP_H, CROP_W), jnp.int32),
    ),
    mesh=_mesh,
    scratch_types=[
        pltpu.VMEM((16,), jnp.int32),
        pltpu.VMEM((SR, WB), jnp.float32),
        pltpu.VMEM((RPW, CROP_W), jnp.float32),
        pltpu.VMEM((RPW, CROP_W), jnp.int32),
    ],
    compiler_params=_params,
)
def _crop_kernel(img_hbm, labf_hbm, sel_hbm, oimg_hbm, olab_hbm,
                 sel_v, buf_v, obuf_v, olbuf_v):
    w = lax.axis_index("s") * NC + lax.axis_index("c")
    iota = lax.iota(jnp.int32, 16)

    @pl.when(w < ACT)
    def _():
        pltpu.sync_copy(sel_hbm, sel_v)
        sv = sel_v[...]
        top = _pick(sv, iota, 0)
        left = _pick(sv, iota, 1)
        rs = top & 7
        lb = pl.multiple_of(jnp.minimum(left & -128, W - WB), 128)
        shift = left - lb
        base8 = (top & -8) + RPW * w
        start = pl.multiple_of(jnp.minimum(base8, ROW_CLAMP), 8)
        roff = base8 - start + rs
        olo = pl.multiple_of(RPW * w, 8)
        cbase = shift + iota

        def shift_rows(as_label):
            def body(i, carry):
                rowv = jnp.full((16,), roff + i)
                for g in range(0, CROP_W // 16, 8):
                    vs = [plsc.load_gather(
                        buf_v, [rowv, cbase + ((g + k) * 16)])
                        for k in range(8)]
                    for k in range(8):
                        if as_label:
                            olbuf_v[i, pl.ds((g + k) * 16, 16)] = (
                                plsc.bitcast(vs[k], jnp.int32))
                        else:
                            obuf_v[i, pl.ds((g + k) * 16, 16)] = vs[k]
                return carry
            lax.fori_loop(0, RPW, body, 0)

        for ch in range(3):
            pltpu.sync_copy(
                img_hbm.at[ch, pl.ds(start, SR), pl.ds(lb, WB)], buf_v)
            shift_rows(False)
            pltpu.sync_copy(obuf_v, oimg_hbm.at[ch, pl.ds(olo, RPW), :])

        pltpu.sync_copy(
            labf_hbm.at[pl.ds(start, SR), pl.ds(lb, WB)], buf_v)
        shift_rows(True)
        pltpu.sync_copy(olbuf_v, olab_hbm.at[pl.ds(olo, RPW), :])


def kernel(image, label_image, label_costs):
    label2d = label_image.reshape(H, W)
    label_f = jax.lax.bitcast_convert_type(label2d, jnp.float32)

    parts = _hist_kernel(label2d)
    counts = parts.reshape(ACT, SAMPLES, CROP_PAD)[:, :, :LABEL_COUNT * 64]
    counts = counts.reshape(ACT, SAMPLES, LABEL_COUNT, 64).sum(axis=(0, 3))

    norm_costs = label_costs / jnp.maximum(jnp.sum(jnp.abs(label_costs)), 1e-12)
    total = float(CROP_H * CROP_W)

    def cost_of(c):
        dist = counts[c].astype(jnp.float32) / total
        return jnp.sum(norm_costs * dist)

    best_cost = cost_of(0)
    best_idx = jnp.int32(0)
    for c in range(1, SAMPLES):
        cc = cost_of(c)
        better = cc < best_cost
        best_idx = jnp.where(better, jnp.int32(c), best_idx)
        best_cost = jnp.where(better, cc, best_cost)

    tops_a = jnp.asarray(_TOPS, jnp.int32)
    lefts_a = jnp.asarray(_LEFTS, jnp.int32)
    sel = jnp.zeros((16,), jnp.int32)
    sel = sel.at[0].set(tops_a[best_idx]).at[1].set(lefts_a[best_idx])

    best_image, best_label = _crop_kernel(image, label_f, sel)
    return best_image, best_label.reshape(1, CROP_H, CROP_W), best_cost


# speedup vs baseline: 1.0972x; 1.0972x over previous
import functools

import jax
import jax.numpy as jnp
from jax import lax
from jax.experimental import pallas as pl
from jax.experimental.pallas import tpu as pltpu
from jax.experimental.pallas import tpu_sc as plsc

H, W = 1024, 2048
CROP_H, CROP_W = 720, 1280
SAMPLES = 4
LABEL_COUNT = 19
NC, NS = 2, 16
NWORK = NC * NS
ACT = 30
RPW = CROP_H // ACT
SR = 32
WB = 1408
NVEC = WB // 16
NBANK = 4
CROP_PAD = 1280
HR = SAMPLES * CROP_PAD // 128
ROW_CLAMP = H - SR

_TOPS = (219, 196, 73, 29)
_LEFTS = (192, 367, 42, 696)

_mesh = plsc.VectorSubcoreMesh(core_axis_name="c", subcore_axis_name="s")
_params = pltpu.CompilerParams(
    use_tc_tiling_on_sc=True, needs_layout_passes=False)


def _pick(vec, iota, k):
    return jnp.max(jnp.where(iota == k, vec, 0))


def _col_masks(shift, iota_np):
    import numpy as np
    masks = []
    for j in range(NVEC):
        cols = iota_np + 16 * j
        m = (cols >= shift) & (cols < shift + CROP_W)
        masks.append(m)
    return masks


@functools.partial(
    pl.kernel,
    out_type=jax.ShapeDtypeStruct((ACT * SAMPLES * CROP_PAD,), jnp.int32),
    mesh=_mesh,
    scratch_types=[
        pltpu.VMEM((SR, WB), jnp.int32),
        pltpu.VMEM((SR, WB), jnp.int32),
        pltpu.VMEM((SAMPLES * CROP_PAD,), jnp.int32),
        pltpu.SemaphoreType.DMA,
        pltpu.SemaphoreType.DMA,
    ],
    compiler_params=_params,
)
def _hist_kernel(label_hbm, out_hbm, buf0_v, buf1_v, hist_v, sem0, sem1):
    import numpy as np
    w = lax.axis_index("s") * NC + lax.axis_index("c")
    iota = lax.iota(jnp.int32, 16)
    iota_np = np.arange(16)
    zeros = jnp.zeros((16,), jnp.int32)
    ones = jnp.ones((16,), jnp.int32)
    bufs = (buf0_v, buf1_v)
    sems = (sem0, sem1)

    @pl.when(w < ACT)
    def _():
        for k in range(SAMPLES * CROP_PAD // 16):
            hist_v[pl.ds(k * 16, 16)] = zeros

        def src_roff(c):
            top, left = _TOPS[c], _LEFTS[c]
            lb = min(left & -128, W - WB)
            base8 = (top & -8) + RPW * w
            start = pl.multiple_of(jnp.minimum(base8, ROW_CLAMP), 8)
            roff = base8 - start + (top & 7)
            return label_hbm.at[pl.ds(start, SR), lb:lb + WB], roff

        src0, roff0 = src_roff(0)
        pend = pltpu.async_copy(src0, bufs[0], sems[0])
        roffs = {0: roff0}
        for c in range(SAMPLES):
            pend.wait()
            if c + 1 < SAMPLES:
                srcn, roffs[c + 1] = src_roff(c + 1)
                pend = pltpu.async_copy(
                    srcn, bufs[(c + 1) % 2], sems[(c + 1) % 2])

            top, left = _TOPS[c], _LEFTS[c]
            lb = min(left & -128, W - WB)
            shift = left - lb
            masks_np = _col_masks(shift, iota_np)
            js = [j for j in range(NVEC) if masks_np[j].any()]
            buf_v = bufs[c % 2]
            roff = roffs[c]

            def body(i, carry, _c=c, _js=js, _masks=masks_np,
                     _roff=roff, _buf=buf_v, _shift=shift):
                r = _roff + i
                for g in range(0, len(_js), 8):
                    grp = _js[g:g + 8]
                    idxs, ms = [], []
                    for j in grp:
                        lv = _buf[r, pl.ds(j * 16, 16)]
                        bank = (j % NBANK) * 16
                        idxs.append(lv * (16 * NBANK)
                                    + (_c * CROP_PAD + bank + iota))
                        if _masks[j].all():
                            ms.append(None)
                        else:
                            cols16 = iota + (16 * j)
                            ms.append(jnp.logical_and(
                                cols16 >= _shift, cols16 < _shift + CROP_W))
                    for k in range(len(grp)):
                        plsc.addupdate_scatter(
                            hist_v, [idxs[k]], ones, mask=ms[k])
                return carry

            lax.fori_loop(0, RPW, body, 0)

        pltpu.sync_copy(
            hist_v,
            out_hbm.at[pl.ds(
                pl.multiple_of(SAMPLES * CROP_PAD * w, 8),
                SAMPLES * CROP_PAD)])


@functools.partial(
    pl.kernel,
    out_type=(
        jax.ShapeDtypeStruct((3, CROP_H, CROP_W), jnp.float32),
        jax.ShapeDtypeStruct((CROP_H, CROP_W), jnp.int32),
    ),
    mesh=_mesh,
    scratch_types=[
        pltpu.VMEM((16,), jnp.int32),
        pltpu.VMEM((SR, WB), jnp.float32),
        pltpu.VMEM((RPW, CROP_W), jnp.float32),
        pltpu.VMEM((RPW, CROP_W), jnp.int32),
    ],
    compiler_params=_params,
)
def _crop_kernel(img_hbm, labf_hbm, sel_hbm, oimg_hbm, olab_hbm,
                 sel_v, buf_v, obuf_v, olbuf_v):
    w = lax.axis_index("s") * NC + lax.axis_index("c")
    iota = lax.iota(jnp.int32, 16)

    @pl.when(w < ACT)
    def _():
        pltpu.sync_copy(sel_hbm, sel_v)
        sv = sel_v[...]
        top = _pick(sv, iota, 0)
        left = _pick(sv, iota, 1)
        rs = top & 7
        lb = pl.multiple_of(jnp.minimum(left & -128, W - WB), 128)
        shift = left - lb
        base8 = (top & -8) + RPW * w
        start = pl.multiple_of(jnp.minimum(base8, ROW_CLAMP), 8)
        roff = base8 - start + rs
        olo = pl.multiple_of(RPW * w, 8)
        cbase = shift + iota

        def shift_rows(as_label):
            def body(i, carry):
                rowv = jnp.full((16,), roff + i)
                for g in range(0, CROP_W // 16, 8):
                    vs = [plsc.load_gather(
                        buf_v, [rowv, cbase + ((g + k) * 16)])
                        for k in range(8)]
                    for k in range(8):
                        if as_label:
                            olbuf_v[i, pl.ds((g + k) * 16, 16)] = (
                                plsc.bitcast(vs[k], jnp.int32))
                        else:
                            obuf_v[i, pl.ds((g + k) * 16, 16)] = vs[k]
                return carry
            lax.fori_loop(0, RPW, body, 0)

        for ch in range(3):
            pltpu.sync_copy(
                img_hbm.at[ch, pl.ds(start, SR), pl.ds(lb, WB)], buf_v)
            shift_rows(False)
            pltpu.sync_copy(obuf_v, oimg_hbm.at[ch, pl.ds(olo, RPW), :])

        pltpu.sync_copy(
            labf_hbm.at[pl.ds(start, SR), pl.ds(lb, WB)], buf_v)
        shift_rows(True)
        pltpu.sync_copy(olbuf_v, olab_hbm.at[pl.ds(olo, RPW), :])


def kernel(image, label_image, label_costs):
    label2d = label_image.reshape(H, W)
    label_f = jax.lax.bitcast_convert_type(label2d, jnp.float32)

    parts = _hist_kernel(label2d)
    counts = parts.reshape(ACT, SAMPLES, CROP_PAD)[:, :, :LABEL_COUNT * 64]
    counts = counts.reshape(ACT, SAMPLES, LABEL_COUNT, 64).sum(axis=(0, 3))

    norm_costs = label_costs / jnp.maximum(jnp.sum(jnp.abs(label_costs)), 1e-12)
    total = float(CROP_H * CROP_W)

    def cost_of(c):
        dist = counts[c].astype(jnp.float32) / total
        return jnp.sum(norm_costs * dist)

    best_cost = cost_of(0)
    best_idx = jnp.int32(0)
    for c in range(1, SAMPLES):
        cc = cost_of(c)
        better = cc < best_cost
        best_idx = jnp.where(better, jnp.int32(c), best_idx)
        best_cost = jnp.where(better, cc, best_cost)

    tops_a = jnp.asarray(_TOPS, jnp.int32)
    lefts_a = jnp.asarray(_LEFTS, jnp.int32)
    sel = jnp.zeros((16,), jnp.int32)
    sel = sel.at[0].set(tops_a[best_idx]).at[1].set(lefts_a[best_idx])

    best_image, best_label = _crop_kernel(image, label_f, sel)
    return best_image, best_label.reshape(1, CROP_H, CROP_W), best_cost


# speedup vs baseline: 1.1569x; 1.0544x over previous
import functools

import jax
import jax.numpy as jnp
from jax import lax
from jax.experimental import pallas as pl
from jax.experimental.pallas import tpu as pltpu
from jax.experimental.pallas import tpu_sc as plsc

H, W = 1024, 2048
CROP_H, CROP_W = 720, 1280
SAMPLES = 4
LABEL_COUNT = 19
NC, NS = 2, 16
NWORK = NC * NS
ACT = 30
RPW = CROP_H // ACT
SR = 32
WB = 1408
NVEC = WB // 16
NBANK = 4
CROP_PAD = 1280
HR = SAMPLES * CROP_PAD // 128
ROW_CLAMP = H - SR

_TOPS = (219, 196, 73, 29)
_LEFTS = (192, 367, 42, 696)

_mesh = plsc.VectorSubcoreMesh(core_axis_name="c", subcore_axis_name="s")
_params = pltpu.CompilerParams(
    use_tc_tiling_on_sc=True, needs_layout_passes=False)


def _pick(vec, iota, k):
    return jnp.max(jnp.where(iota == k, vec, 0))


def _col_masks(shift, iota_np):
    import numpy as np
    masks = []
    for j in range(NVEC):
        cols = iota_np + 16 * j
        m = (cols >= shift) & (cols < shift + CROP_W)
        masks.append(m)
    return masks


@functools.partial(
    pl.kernel,
    out_type=jax.ShapeDtypeStruct((ACT * SAMPLES * CROP_PAD,), jnp.int32),
    mesh=_mesh,
    scratch_types=[
        pltpu.VMEM((SR, WB), jnp.int32),
        pltpu.VMEM((SR, WB), jnp.int32),
        pltpu.VMEM((SAMPLES * CROP_PAD,), jnp.int32),
        pltpu.SemaphoreType.DMA,
        pltpu.SemaphoreType.DMA,
    ],
    compiler_params=_params,
)
def _hist_kernel(label_hbm, out_hbm, buf0_v, buf1_v, hist_v, sem0, sem1):
    import numpy as np
    w = lax.axis_index("s") * NC + lax.axis_index("c")
    iota = lax.iota(jnp.int32, 16)
    iota_np = np.arange(16)
    zeros = jnp.zeros((16,), jnp.int32)
    ones = jnp.ones((16,), jnp.int32)
    bufs = (buf0_v, buf1_v)
    sems = (sem0, sem1)

    @pl.when(w < ACT)
    def _():
        for k in range(SAMPLES * CROP_PAD // 16):
            hist_v[pl.ds(k * 16, 16)] = zeros

        def src_roff(c):
            top, left = _TOPS[c], _LEFTS[c]
            lb = min(left & -128, W - WB)
            base8 = (top & -8) + RPW * w
            start = pl.multiple_of(jnp.minimum(base8, ROW_CLAMP), 8)
            roff = base8 - start + (top & 7)
            return label_hbm.at[pl.ds(start, SR), lb:lb + WB], roff

        src0, roff0 = src_roff(0)
        pend = pltpu.async_copy(src0, bufs[0], sems[0])
        roffs = {0: roff0}
        for c in range(SAMPLES):
            pend.wait()
            if c + 1 < SAMPLES:
                srcn, roffs[c + 1] = src_roff(c + 1)
                pend = pltpu.async_copy(
                    srcn, bufs[(c + 1) % 2], sems[(c + 1) % 2])

            top, left = _TOPS[c], _LEFTS[c]
            lb = min(left & -128, W - WB)
            shift = left - lb
            masks_np = _col_masks(shift, iota_np)
            js = [j for j in range(NVEC) if masks_np[j].any()]
            buf_v = bufs[c % 2]
            roff = roffs[c]

            def body(i, carry, _c=c, _js=js, _masks=masks_np,
                     _roff=roff, _buf=buf_v, _shift=shift):
                r = _roff + i
                for g in range(0, len(_js), 8):
                    grp = _js[g:g + 8]
                    idxs, ms = [], []
                    for j in grp:
                        lv = _buf[r, pl.ds(j * 16, 16)]
                        bank = (j % NBANK) * 16
                        idxs.append(lv * (16 * NBANK)
                                    + (_c * CROP_PAD + bank + iota))
                        if _masks[j].all():
                            ms.append(None)
                        else:
                            cols16 = iota + (16 * j)
                            ms.append(jnp.logical_and(
                                cols16 >= _shift, cols16 < _shift + CROP_W))
                    for k in range(len(grp)):
                        plsc.addupdate_scatter(
                            hist_v, [idxs[k]], ones, mask=ms[k])
                return carry

            lax.fori_loop(0, RPW, body, 0)

        pltpu.sync_copy(
            hist_v,
            out_hbm.at[pl.ds(
                pl.multiple_of(SAMPLES * CROP_PAD * w, 8),
                SAMPLES * CROP_PAD)])


@functools.partial(
    pl.kernel,
    out_type=(
        jax.ShapeDtypeStruct((3, CROP_H, CROP_W), jnp.float32),
        jax.ShapeDtypeStruct((CROP_H, CROP_W), jnp.float32),
    ),
    mesh=_mesh,
    scratch_types=[
        pltpu.VMEM((16,), jnp.int32),
        pltpu.VMEM((SR, WB), jnp.float32),
        pltpu.VMEM((SR, WB), jnp.float32),
        pltpu.VMEM((RPW, CROP_W), jnp.float32),
        pltpu.SemaphoreType.DMA,
        pltpu.SemaphoreType.DMA,
    ],
    compiler_params=_params,
)
def _crop_kernel(img_hbm, labf_hbm, sel_hbm, oimg_hbm, olabf_hbm,
                 sel_v, buf0_v, buf1_v, obuf_v, sem0, sem1):
    w = lax.axis_index("s") * NC + lax.axis_index("c")
    iota = lax.iota(jnp.int32, 16)
    bufs = (buf0_v, buf1_v)
    sems = (sem0, sem1)

    @pl.when(w < ACT)
    def _():
        pltpu.sync_copy(sel_hbm, sel_v)
        sv = sel_v[...]
        top = _pick(sv, iota, 0)
        left = _pick(sv, iota, 1)
        rs = top & 7
        lb = pl.multiple_of(jnp.minimum(left & -128, W - WB), 128)
        shift = left - lb
        base8 = (top & -8) + RPW * w
        start = pl.multiple_of(jnp.minimum(base8, ROW_CLAMP), 8)
        roff = base8 - start + rs
        olo = pl.multiple_of(RPW * w, 8)
        cbase = shift + iota

        def src(t):
            if t < 3:
                return img_hbm.at[t, pl.ds(start, SR), pl.ds(lb, WB)]
            return labf_hbm.at[pl.ds(start, SR), pl.ds(lb, WB)]

        def shift_rows(buf_v):
            def body(i, carry):
                rowv = jnp.full((16,), roff + i)
                for g in range(0, CROP_W // 16, 8):
                    vs = [plsc.load_gather(
                        buf_v, [rowv, cbase + ((g + k) * 16)])
                        for k in range(8)]
                    for k in range(8):
                        obuf_v[i, pl.ds((g + k) * 16, 16)] = vs[k]
                return carry
            lax.fori_loop(0, RPW, body, 0)

        pend = pltpu.async_copy(src(0), bufs[0], sems[0])
        for t in range(4):
            pend.wait()
            if t + 1 < 4:
                pend = pltpu.async_copy(
                    src(t + 1), bufs[(t + 1) % 2], sems[(t + 1) % 2])
            shift_rows(bufs[t % 2])
            if t < 3:
                pltpu.sync_copy(obuf_v, oimg_hbm.at[t, pl.ds(olo, RPW), :])
            else:
                pltpu.sync_copy(obuf_v, olabf_hbm.at[pl.ds(olo, RPW), :])


def kernel(image, label_image, label_costs):
    label2d = label_image.reshape(H, W)
    label_f = jax.lax.bitcast_convert_type(label2d, jnp.float32)

    parts = _hist_kernel(label2d)
    counts = parts.reshape(ACT, SAMPLES, CROP_PAD)[:, :, :LABEL_COUNT * 64]
    counts = counts.reshape(ACT, SAMPLES, LABEL_COUNT, 64).sum(axis=(0, 3))

    norm_costs = label_costs / jnp.maximum(jnp.sum(jnp.abs(label_costs)), 1e-12)
    total = float(CROP_H * CROP_W)

    def cost_of(c):
        dist = counts[c].astype(jnp.float32) / total
        return jnp.sum(norm_costs * dist)

    best_cost = cost_of(0)
    best_idx = jnp.int32(0)
    for c in range(1, SAMPLES):
        cc = cost_of(c)
        better = cc < best_cost
        best_idx = jnp.where(better, jnp.int32(c), best_idx)
        best_cost = jnp.where(better, cc, best_cost)

    tops_a = jnp.asarray(_TOPS, jnp.int32)
    lefts_a = jnp.asarray(_LEFTS, jnp.int32)
    sel = jnp.zeros((16,), jnp.int32)
    sel = sel.at[0].set(tops_a[best_idx]).at[1].set(lefts_a[best_idx])

    best_image, best_label_f = _crop_kernel(image, label_f, sel)
    best_label = jax.lax.bitcast_convert_type(best_label_f, jnp.int32)
    return best_image, best_label.reshape(1, CROP_H, CROP_W), best_cost
